# Initial kernel scaffold; baseline (speedup 1.0000x reference)
import jax
import jax.numpy as jnp
from jax.experimental import pallas as pl


def kernel(x, b1_w, b1_b, b1_bn_gamma, b1_bn_beta, b1_bn_mean, b1_bn_var, b2_dw_w, b2_dw_b, b2_pw_w, b2_pw_b, b2_bn_gamma, b2_bn_beta, b2_bn_mean, b2_bn_var, b3_dw_w, b3_dw_b, b3_pw_w, b3_pw_b, b3_bn_gamma, b3_bn_beta, b3_bn_mean, b3_bn_var, b4_dw_w, b4_dw_b, b4_pw_w, b4_pw_b, b4_bn_gamma, b4_bn_beta, b4_bn_mean, b4_bn_var, gf_w, gf_b, gf_bn_gamma, gf_bn_beta, gf_bn_mean, gf_bn_var, gc_w, gc_b, gc_bn_gamma, gc_bn_beta, gc_bn_mean, gc_bn_var, bc_w, bc_b, bc_bn_gamma, bc_bn_beta, bc_bn_mean, bc_bn_var, ba_w, ba_b, ba_bn_gamma, ba_bn_beta, ba_bn_mean, ba_bn_var, qkv_w, proj_dw_w, proj_pw_w, proj_bn_gamma, proj_bn_beta, proj_bn_mean, proj_bn_var, rel_pos_table):
    raise NotImplementedError("write your pallas kernel here")


# speedup vs baseline: 2.0311x; 2.0311x over previous
import jax
import jax.numpy as jnp
from jax import lax
from jax.experimental import pallas as pl
from jax.experimental.pallas import tpu as pltpu

BN_EPS = 1e-5
BF = jnp.bfloat16
F32 = jnp.float32

WS = 8
NH = 4


def _bn_affine(g, be, m, v):
    s = g / jnp.sqrt(v + BN_EPS)
    return s, be - m * s


def _rel_bias(rel_pos_table, ws, nh):
    coords = jnp.stack(jnp.meshgrid(jnp.arange(ws), jnp.arange(ws),
                                    indexing="ij"))
    flat = coords.reshape(2, -1)
    rel = flat[:, :, None] - flat[:, None, :]
    rel = rel.transpose(1, 2, 0) + jnp.array([ws - 1, ws - 1])
    idx = (rel[:, :, 0] * (2 * ws - 1) + rel[:, :, 1]).reshape(-1)
    L = ws * ws
    return rel_pos_table[idx].reshape(L, L, nh).transpose(2, 0, 1).astype(F32)


def _make_body(H, W, C, ws, nh):
    HW = H * W
    L = ws * ws
    HD = C // nh
    hh, ww = H // ws, W // ws
    NWIN = hh * ww

    def body(x_ref, dw2_ref, dw3_ref, dw4_ref, wb_ref, sb_ref, tb_ref,
             wc_ref, sc_ref, tc_ref, wq_ref,
             gfw_ref, gfs_ref, gft_ref, gcw_ref, gcs_ref, gct_ref,
             wcs_ref, bcs_ref, wa_ref, sa_ref, ta_ref,
             bias_ref, dwp_ref, we_ref, se_ref, o_ref):
        x_cm = x_ref[0]
        xbf = x_cm.astype(BF)
        x2 = jnp.transpose(xbf)
        x3 = x2.reshape(H, W, C)

        P = 4
        zr = jnp.zeros((P, W, C), BF)
        xp = jnp.concatenate([zr, x3, zr], axis=0)
        zc = jnp.zeros((H + 2 * P, P, C), BF)
        xp = jnp.concatenate([zc, xp, zc], axis=1)

        def dconv(w_ref, k, dl, p):
            acc = None
            t = 0
            for i in range(k):
                for j in range(k):
                    dy, dx = i * dl - p, j * dl - p
                    tap = xp[P + dy:P + dy + H, P + dx:P + dx + W, :]
                    term = tap.astype(F32) * w_ref[t]
                    acc = term if acc is None else acc + term
                    t += 1
            return acc.astype(BF).reshape(HW, C)

        y2 = dconv(dw2_ref, 3, 1, 1)
        y3 = dconv(dw3_ref, 4, 2, 3)
        y4 = dconv(dw4_ref, 5, 2, 4)

        def branch(inp, idx):
            y = jnp.dot(inp, wb_ref[idx], preferred_element_type=F32)
            y = y * sb_ref[idx] + tb_ref[idx]
            return jnp.clip(y, 0.0, 6.0).astype(BF)

        acc = jnp.dot(branch(x2, 0), wc_ref[0], preferred_element_type=F32)
        acc += jnp.dot(branch(y2, 1), wc_ref[1], preferred_element_type=F32)
        acc += jnp.dot(branch(y3, 2), wc_ref[2], preferred_element_type=F32)
        acc += jnp.dot(branch(y4, 3), wc_ref[3], preferred_element_type=F32)

        gmean = jnp.mean(x2.astype(F32), axis=0, keepdims=True)
        gf = jnp.dot(gmean, gfw_ref[...], preferred_element_type=F32)
        gf = jnp.clip(gf * gfs_ref[...] + gft_ref[...], 0.0, 6.0)
        gc = jnp.dot(gf, gcw_ref[...], preferred_element_type=F32)
        gc = jnp.clip(gc * gcs_ref[...] + gct_ref[...], 0.0, 6.0)
        gsh = (jnp.dot(gc, wcs_ref[...], preferred_element_type=F32)
               + bcs_ref[...]) * sc_ref[...] + tc_ref[...]

        br = jnp.clip(acc * sc_ref[...] + gsh, 0.0, 6.0).astype(BF)
        loc = jnp.dot(br, wa_ref[...], preferred_element_type=F32)
        loc = jnp.clip(loc * sa_ref[...] + ta_ref[...], 0.0, 6.0)

        qkv = jnp.dot(x2, wq_ref[...], preferred_element_type=F32).astype(BF)
        wm = (qkv.reshape(hh, ws, ww, ws, 3 * C)
              .transpose(0, 2, 1, 3, 4).reshape(NWIN, L, 3 * C))
        outs = []
        for h in range(nh):
            q = wm[:, :, h * HD:(h + 1) * HD].astype(F32)
            k = wm[:, :, C + h * HD:C + (h + 1) * HD].astype(F32)
            v = wm[:, :, 2 * C + h * HD:2 * C + (h + 1) * HD].astype(F32)
            dots = lax.dot_general(q, k, (((2,), (2,)), ((0,), (0,))),
                                   preferred_element_type=F32)
            dots = dots + bias_ref[h]
            m = jnp.max(dots, axis=-1, keepdims=True)
            p = jnp.exp(dots - m)
            s = jnp.sum(p, axis=-1, keepdims=True)
            p = p * pl.reciprocal(s, approx=True)
            outs.append(lax.dot_general(p, v, (((2,), (1,)), ((0,), (0,))),
                                        preferred_element_type=F32))
        aw = jnp.concatenate(outs, axis=-1)
        attn = (aw.reshape(hh, ww, ws, ws, C)
                .transpose(0, 2, 1, 3, 4).reshape(H, W, C))

        ph = ws // 2 - 1
        z3r = jnp.zeros((ph, W, C), F32)
        app = jnp.concatenate([z3r, attn, attn[H - 2:H - 1], z3r], axis=0)
        ax = app[0:H]
        for t in range(1, ws):
            ax = ax + app[t:t + H]
        z3c = jnp.zeros((H, ph, C), F32)
        apw = jnp.concatenate([z3c, attn, attn[:, W - 2:W - 1], z3c], axis=1)
        ay = apw[:, 0:W]
        for t in range(1, ws):
            ay = ay + apw[:, t:t + W]
        out = ax * 0.125 + ay * 0.125 + loc.reshape(H, W, C)

        outp = jnp.concatenate([out, out[H - 2:H - 1]], axis=0)
        outp = jnp.concatenate([outp, outp[:, W - 2:W - 1]], axis=1)
        pp = (ws - 1) // 2
        zpr = jnp.zeros((pp, W + 1, C), F32)
        op = jnp.concatenate([zpr, outp, zpr], axis=0)
        zpc = jnp.zeros((H + 1 + 2 * pp, pp, C), F32)
        op = jnp.concatenate([zpc, op, zpc], axis=1)
        dacc = None
        for i in range(ws):
            for j in range(ws):
                term = op[i:i + H, j:j + W, :] * dwp_ref[i * ws + j]
                dacc = term if dacc is None else dacc + term
        dwb = dacc.astype(BF).reshape(HW, C)

        ycm = lax.dot_general(we_ref[...], dwb, (((0,), (1,)), ((), ())),
                              preferred_element_type=F32)
        o_ref[0] = ycm + se_ref[...]

    return body


def kernel(x, b1_w, b1_b, b1_bn_gamma, b1_bn_beta, b1_bn_mean, b1_bn_var,
           b2_dw_w, b2_dw_b, b2_pw_w, b2_pw_b,
           b2_bn_gamma, b2_bn_beta, b2_bn_mean, b2_bn_var,
           b3_dw_w, b3_dw_b, b3_pw_w, b3_pw_b,
           b3_bn_gamma, b3_bn_beta, b3_bn_mean, b3_bn_var,
           b4_dw_w, b4_dw_b, b4_pw_w, b4_pw_b,
           b4_bn_gamma, b4_bn_beta, b4_bn_mean, b4_bn_var,
           gf_w, gf_b, gf_bn_gamma, gf_bn_beta, gf_bn_mean, gf_bn_var,
           gc_w, gc_b, gc_bn_gamma, gc_bn_beta, gc_bn_mean, gc_bn_var,
           bc_w, bc_b, bc_bn_gamma, bc_bn_beta, bc_bn_mean, bc_bn_var,
           ba_w, ba_b, ba_bn_gamma, ba_bn_beta, ba_bn_mean, ba_bn_var,
           qkv_w,
           proj_dw_w, proj_pw_w,
           proj_bn_gamma, proj_bn_beta, proj_bn_mean, proj_bn_var,
           rel_pos_table):
    B, C, H, W = x.shape
    ws, nh = WS, NH
    HW = H * W
    L = ws * ws
    scale = (C // nh) ** -0.5

    s1, t1 = _bn_affine(b1_bn_gamma, b1_bn_beta, b1_bn_mean, b1_bn_var)
    t1 = t1 + b1_b * s1
    s2, t2 = _bn_affine(b2_bn_gamma, b2_bn_beta, b2_bn_mean, b2_bn_var)
    t2 = t2 + b2_pw_b * s2 + (b2_dw_b @ b2_pw_w) * s2
    s3, t3 = _bn_affine(b3_bn_gamma, b3_bn_beta, b3_bn_mean, b3_bn_var)
    t3 = t3 + b3_pw_b * s3 + (b3_dw_b @ b3_pw_w) * s3
    s4, t4 = _bn_affine(b4_bn_gamma, b4_bn_beta, b4_bn_mean, b4_bn_var)
    t4 = t4 + b4_pw_b * s4 + (b4_dw_b @ b4_pw_w) * s4

    wb = jnp.stack([b1_w, b2_pw_w, b3_pw_w, b4_pw_w], 0).astype(BF)
    sb = jnp.stack([s1, s2, s3, s4], 0).reshape(4, 1, C)
    tb = jnp.stack([t1, t2, t3, t4], 0).reshape(4, 1, C)

    wc_s = bc_w[:, 0::4]
    bc_s = bc_b[0::4]
    sc, tc = _bn_affine(bc_bn_gamma[0::4], bc_bn_beta[0::4],
                        bc_bn_mean[0::4], bc_bn_var[0::4])
    wc = wc_s.reshape(4, C, C).astype(BF)

    sa, ta = _bn_affine(ba_bn_gamma, ba_bn_beta, ba_bn_mean, ba_bn_var)
    ta = ta + ba_b * sa

    sgf, tgf = _bn_affine(gf_bn_gamma, gf_bn_beta, gf_bn_mean, gf_bn_var)
    tgf = tgf + gf_b * sgf
    sgc, tgc = _bn_affine(gc_bn_gamma, gc_bn_beta, gc_bn_mean, gc_bn_var)
    tgc = tgc + gc_b * sgc

    wq = jnp.concatenate([qkv_w[:, :C] * scale, qkv_w[:, C:]], 1).astype(BF)

    sp, tp = _bn_affine(proj_bn_gamma, proj_bn_beta, proj_bn_mean,
                        proj_bn_var)
    w_eff = (sp[:, None] * proj_pw_w).astype(BF)
    shift_eff = (tp @ proj_pw_w).reshape(C, 1)

    bias = _rel_bias(rel_pos_table, ws, nh)

    dw2 = b2_dw_w.reshape(9, C)
    dw3 = b3_dw_w.reshape(16, C)
    dw4 = b4_dw_w.reshape(25, C)
    dwp = proj_dw_w.reshape(ws * ws, C)

    args = [x.reshape(B, C, HW),
            dw2, dw3, dw4, wb, sb, tb, wc,
            sc.reshape(1, C), tc.reshape(1, C), wq,
            gf_w, sgf.reshape(1, C), tgf.reshape(1, C),
            gc_w, sgc.reshape(1, 4 * C), tgc.reshape(1, 4 * C),
            wc_s, bc_s.reshape(1, C), ba_w.astype(BF),
            sa.reshape(1, C), ta.reshape(1, C),
            bias, dwp, w_eff, shift_eff]

    def const_spec(a):
        zeros = (0,) * a.ndim
        return pl.BlockSpec(a.shape, lambda b, _z=zeros: _z)

    in_specs = [pl.BlockSpec((1, C, HW), lambda b: (b, 0, 0))]
    in_specs += [const_spec(a) for a in args[1:]]

    out = pl.pallas_call(
        _make_body(H, W, C, ws, nh),
        out_shape=jax.ShapeDtypeStruct((B, C, HW), F32),
        grid=(B,),
        in_specs=in_specs,
        out_specs=pl.BlockSpec((1, C, HW), lambda b: (b, 0, 0)),
        compiler_params=pltpu.CompilerParams(
            dimension_semantics=("parallel",),
            vmem_limit_bytes=60 * 1024 * 1024),
    )(*args)
    return out.reshape(B, C, H, W)


# speedup vs baseline: 2.0583x; 1.0134x over previous
import jax
import jax.numpy as jnp
from jax import lax
from jax.experimental import pallas as pl
from jax.experimental.pallas import tpu as pltpu

BN_EPS = 1e-5
BF = jnp.bfloat16
F32 = jnp.float32

WS = 8
NH = 4


def _bn_affine(g, be, m, v):
    s = g / jnp.sqrt(v + BN_EPS)
    return s, be - m * s


def _rel_bias(rel_pos_table, ws, nh):
    coords = jnp.stack(jnp.meshgrid(jnp.arange(ws), jnp.arange(ws),
                                    indexing="ij"))
    flat = coords.reshape(2, -1)
    rel = flat[:, :, None] - flat[:, None, :]
    rel = rel.transpose(1, 2, 0) + jnp.array([ws - 1, ws - 1])
    idx = (rel[:, :, 0] * (2 * ws - 1) + rel[:, :, 1]).reshape(-1)
    L = ws * ws
    return rel_pos_table[idx].reshape(L, L, nh).transpose(2, 0, 1).astype(F32)


def _make_body(H, W, C, ws, nh):
    HW = H * W
    L = ws * ws
    HD = C // nh
    hh, ww = H // ws, W // ws
    NWIN = hh * ww

    def body(x_ref, dw2_ref, dw3_ref, dw4_ref, wb_ref, sb_ref, tb_ref,
             wc_ref, sc_ref, tc_ref, wq_ref,
             gfw_ref, gfs_ref, gft_ref, gcw_ref, gcs_ref, gct_ref,
             wcs_ref, bcs_ref, wa_ref, sa_ref, ta_ref,
             bias_ref, dwp_ref, we_ref, se_ref, o_ref):
        x_cm = x_ref[0]
        xbf = x_cm.astype(BF)
        x2 = jnp.transpose(xbf)
        x3 = x2.reshape(H, W, C)

        P = 4
        zr = jnp.zeros((P, W, C), BF)
        xp = jnp.concatenate([zr, x3, zr], axis=0)
        zc = jnp.zeros((H + 2 * P, P, C), BF)
        xp = jnp.concatenate([zc, xp, zc], axis=1)

        def dconv(w_ref, k, dl, p):
            acc = None
            t = 0
            for i in range(k):
                for j in range(k):
                    dy, dx = i * dl - p, j * dl - p
                    tap = xp[P + dy:P + dy + H, P + dx:P + dx + W, :]
                    term = tap.astype(F32) * w_ref[t]
                    acc = term if acc is None else acc + term
                    t += 1
            return acc.astype(BF).reshape(HW, C)

        y2 = dconv(dw2_ref, 3, 1, 1)
        y3 = dconv(dw3_ref, 4, 2, 3)
        y4 = dconv(dw4_ref, 5, 2, 4)

        def branch(inp, idx):
            y = jnp.dot(inp, wb_ref[idx], preferred_element_type=F32)
            y = y * sb_ref[idx] + tb_ref[idx]
            return jnp.clip(y, 0.0, 6.0).astype(BF)

        acc = jnp.dot(branch(x2, 0), wc_ref[0], preferred_element_type=F32)
        acc += jnp.dot(branch(y2, 1), wc_ref[1], preferred_element_type=F32)
        acc += jnp.dot(branch(y3, 2), wc_ref[2], preferred_element_type=F32)
        acc += jnp.dot(branch(y4, 3), wc_ref[3], preferred_element_type=F32)

        gmean = jnp.mean(x2.astype(F32), axis=0, keepdims=True)
        gf = jnp.dot(gmean, gfw_ref[...], preferred_element_type=F32)
        gf = jnp.clip(gf * gfs_ref[...] + gft_ref[...], 0.0, 6.0)
        gc = jnp.dot(gf, gcw_ref[...], preferred_element_type=F32)
        gc = jnp.clip(gc * gcs_ref[...] + gct_ref[...], 0.0, 6.0)
        gsh = (jnp.dot(gc, wcs_ref[...], preferred_element_type=F32)
               + bcs_ref[...]) * sc_ref[...] + tc_ref[...]

        br = jnp.clip(acc * sc_ref[...] + gsh, 0.0, 6.0).astype(BF)
        loc = jnp.dot(br, wa_ref[...], preferred_element_type=F32)
        loc = jnp.clip(loc * sa_ref[...] + ta_ref[...], 0.0, 6.0)

        qkv = jnp.dot(x2, wq_ref[...], preferred_element_type=F32).astype(BF)
        wm = (qkv.reshape(hh, ws, ww, ws, 3 * C)
              .transpose(0, 2, 1, 3, 4).reshape(NWIN, L, 3 * C))
        outs = []
        for h in range(nh):
            q = wm[:, :, h * HD:(h + 1) * HD]
            k = wm[:, :, C + h * HD:C + (h + 1) * HD]
            v = wm[:, :, 2 * C + h * HD:2 * C + (h + 1) * HD]
            dots = lax.dot_general(q, k, (((2,), (2,)), ((0,), (0,))),
                                   preferred_element_type=F32)
            dots = dots + bias_ref[h]
            m = jnp.max(dots, axis=-1, keepdims=True)
            p = jnp.exp(dots - m)
            s = jnp.sum(p, axis=-1, keepdims=True)
            p = (p * pl.reciprocal(s, approx=True)).astype(BF)
            outs.append(lax.dot_general(p, v, (((2,), (1,)), ((0,), (0,))),
                                        preferred_element_type=F32))
        aw = jnp.concatenate(outs, axis=-1)
        attn = (aw.reshape(hh, ww, ws, ws, C)
                .transpose(0, 2, 1, 3, 4).reshape(H, W, C))

        ph = ws // 2 - 1
        z3r = jnp.zeros((ph, W, C), F32)
        app = jnp.concatenate([z3r, attn, attn[H - 2:H - 1], z3r], axis=0)
        ax = app[0:H]
        for t in range(1, ws):
            ax = ax + app[t:t + H]
        z3c = jnp.zeros((H, ph, C), F32)
        apw = jnp.concatenate([z3c, attn, attn[:, W - 2:W - 1], z3c], axis=1)
        ay = apw[:, 0:W]
        for t in range(1, ws):
            ay = ay + apw[:, t:t + W]
        out = ax * 0.125 + ay * 0.125 + loc.reshape(H, W, C)

        outp = jnp.concatenate([out, out[H - 2:H - 1]], axis=0)
        outp = jnp.concatenate([outp, outp[:, W - 2:W - 1]], axis=1)
        pp = (ws - 1) // 2
        zpr = jnp.zeros((pp, W + 1, C), F32)
        op = jnp.concatenate([zpr, outp, zpr], axis=0)
        zpc = jnp.zeros((H + 1 + 2 * pp, pp, C), F32)
        op = jnp.concatenate([zpc, op, zpc], axis=1)
        dacc = None
        for i in range(ws):
            for j in range(ws):
                term = op[i:i + H, j:j + W, :] * dwp_ref[i * ws + j]
                dacc = term if dacc is None else dacc + term
        dwb = dacc.astype(BF).reshape(HW, C)

        ycm = lax.dot_general(we_ref[...], dwb, (((0,), (1,)), ((), ())),
                              preferred_element_type=F32)
        o_ref[0] = ycm + se_ref[...]

    return body


def kernel(x, b1_w, b1_b, b1_bn_gamma, b1_bn_beta, b1_bn_mean, b1_bn_var,
           b2_dw_w, b2_dw_b, b2_pw_w, b2_pw_b,
           b2_bn_gamma, b2_bn_beta, b2_bn_mean, b2_bn_var,
           b3_dw_w, b3_dw_b, b3_pw_w, b3_pw_b,
           b3_bn_gamma, b3_bn_beta, b3_bn_mean, b3_bn_var,
           b4_dw_w, b4_dw_b, b4_pw_w, b4_pw_b,
           b4_bn_gamma, b4_bn_beta, b4_bn_mean, b4_bn_var,
           gf_w, gf_b, gf_bn_gamma, gf_bn_beta, gf_bn_mean, gf_bn_var,
           gc_w, gc_b, gc_bn_gamma, gc_bn_beta, gc_bn_mean, gc_bn_var,
           bc_w, bc_b, bc_bn_gamma, bc_bn_beta, bc_bn_mean, bc_bn_var,
           ba_w, ba_b, ba_bn_gamma, ba_bn_beta, ba_bn_mean, ba_bn_var,
           qkv_w,
           proj_dw_w, proj_pw_w,
           proj_bn_gamma, proj_bn_beta, proj_bn_mean, proj_bn_var,
           rel_pos_table):
    B, C, H, W = x.shape
    ws, nh = WS, NH
    HW = H * W
    L = ws * ws
    scale = (C // nh) ** -0.5

    s1, t1 = _bn_affine(b1_bn_gamma, b1_bn_beta, b1_bn_mean, b1_bn_var)
    t1 = t1 + b1_b * s1
    s2, t2 = _bn_affine(b2_bn_gamma, b2_bn_beta, b2_bn_mean, b2_bn_var)
    t2 = t2 + b2_pw_b * s2 + (b2_dw_b @ b2_pw_w) * s2
    s3, t3 = _bn_affine(b3_bn_gamma, b3_bn_beta, b3_bn_mean, b3_bn_var)
    t3 = t3 + b3_pw_b * s3 + (b3_dw_b @ b3_pw_w) * s3
    s4, t4 = _bn_affine(b4_bn_gamma, b4_bn_beta, b4_bn_mean, b4_bn_var)
    t4 = t4 + b4_pw_b * s4 + (b4_dw_b @ b4_pw_w) * s4

    wb = jnp.stack([b1_w, b2_pw_w, b3_pw_w, b4_pw_w], 0).astype(BF)
    sb = jnp.stack([s1, s2, s3, s4], 0).reshape(4, 1, C)
    tb = jnp.stack([t1, t2, t3, t4], 0).reshape(4, 1, C)

    wc_s = bc_w[:, 0::4]
    bc_s = bc_b[0::4]
    sc, tc = _bn_affine(bc_bn_gamma[0::4], bc_bn_beta[0::4],
                        bc_bn_mean[0::4], bc_bn_var[0::4])
    wc = wc_s.reshape(4, C, C).astype(BF)

    sa, ta = _bn_affine(ba_bn_gamma, ba_bn_beta, ba_bn_mean, ba_bn_var)
    ta = ta + ba_b * sa

    sgf, tgf = _bn_affine(gf_bn_gamma, gf_bn_beta, gf_bn_mean, gf_bn_var)
    tgf = tgf + gf_b * sgf
    sgc, tgc = _bn_affine(gc_bn_gamma, gc_bn_beta, gc_bn_mean, gc_bn_var)
    tgc = tgc + gc_b * sgc

    wq = jnp.concatenate([qkv_w[:, :C] * scale, qkv_w[:, C:]], 1).astype(BF)

    sp, tp = _bn_affine(proj_bn_gamma, proj_bn_beta, proj_bn_mean,
                        proj_bn_var)
    w_eff = (sp[:, None] * proj_pw_w).astype(BF)
    shift_eff = (tp @ proj_pw_w).reshape(C, 1)

    bias = _rel_bias(rel_pos_table, ws, nh)

    dw2 = b2_dw_w.reshape(9, C)
    dw3 = b3_dw_w.reshape(16, C)
    dw4 = b4_dw_w.reshape(25, C)
    dwp = proj_dw_w.reshape(ws * ws, C)

    args = [x.reshape(B, C, HW),
            dw2, dw3, dw4, wb, sb, tb, wc,
            sc.reshape(1, C), tc.reshape(1, C), wq,
            gf_w, sgf.reshape(1, C), tgf.reshape(1, C),
            gc_w, sgc.reshape(1, 4 * C), tgc.reshape(1, 4 * C),
            wc_s, bc_s.reshape(1, C), ba_w.astype(BF),
            sa.reshape(1, C), ta.reshape(1, C),
            bias, dwp, w_eff, shift_eff]

    def const_spec(a):
        zeros = (0,) * a.ndim
        return pl.BlockSpec(a.shape, lambda b, _z=zeros: _z)

    in_specs = [pl.BlockSpec((1, C, HW), lambda b: (b, 0, 0))]
    in_specs += [const_spec(a) for a in args[1:]]

    out = pl.pallas_call(
        _make_body(H, W, C, ws, nh),
        out_shape=jax.ShapeDtypeStruct((B, C, HW), F32),
        grid=(B,),
        in_specs=in_specs,
        out_specs=pl.BlockSpec((1, C, HW), lambda b: (b, 0, 0)),
        compiler_params=pltpu.CompilerParams(
            dimension_semantics=("parallel",),
            vmem_limit_bytes=60 * 1024 * 1024),
    )(*args)
    return out.reshape(B, C, H, W)
